# baseline (speedup 1.0000x reference)
import jax
import jax.numpy as jnp
from jax import lax
from jax.experimental import pallas as pl
from jax.experimental.pallas import tpu as pltpu
from jax.experimental.pallas import tpu_sc as plsc

EMB = 64
B = 16384
PITCH = 128
NB = 1000000

NC = 2
NS = 16
NW = NC * NS
ROWS_PER_W = B // NW
CHUNK = 64
NCHUNK = ROWS_PER_W // CHUNK
NGRP = CHUNK // 16

PAD_R = 4000


def _pad_tbl(tbl, width):
    return jnp.zeros((NB, PITCH), jnp.float32).at[:, :width].set(tbl)


def _sqrt(x):
    xs = jnp.maximum(x, jnp.float32(1e-30))
    i = lax.bitcast_convert_type(xs, jnp.int32)
    i = jnp.int32(0x5F3759DF) - (i >> 1)
    y = lax.bitcast_convert_type(i, jnp.float32)
    for _ in range(3):
        y = y * (jnp.float32(1.5) - jnp.float32(0.5) * xs * y * y)
    return xs * y


def _relu(x):
    return jnp.maximum(x, jnp.float32(0.0))


def _body(cls_hbm, rel_hbm, idx_hbm, out_hbm, idx_v, bufs, out_v, sem):
    wid = lax.axis_index("s") * NC + lax.axis_index("c")
    iota = lax.iota(jnp.int32, 16)

    def chunk_body(k, _):
        base = wid * ROWS_PER_W + k * CHUNK
        ih = [pltpu.async_copy(idx_hbm.at[j, pl.ds(base, CHUNK)],
                               idx_v.at[j], sem) for j in range(13)]
        for h in ih:
            h.wait()

        handles = []
        for j in range(13):
            tbl = cls_hbm if j < 11 else rel_hbm
            handles.append(pltpu.async_copy(
                tbl.at[idx_v.at[j]], bufs.at[j], sem))
        for h in handles:
            h.wait()

        def group_body(g, _):
            rows = g * 16 + iota

            def d_body(dd, a):
                col = jnp.full((16,), dd, dtype=jnp.int32)
                ld = lambda j: plsc.load_gather(bufs.at[j], [rows, col])
                c1, d1 = ld(0), ld(1)
                x1, x2, x3 = ld(2), ld(3), ld(4)
                c3, d3 = ld(5), ld(6)
                c4, d4 = ld(7), ld(8)
                c5, d5 = ld(9), ld(10)
                r3, r4 = ld(11), ld(12)
                t1 = c1 - d1
                u, v, w2 = x2 - x1, x3 - x1, x3 - x2
                s3 = c3 + r3
                t3 = s3 - d3
                s4 = c4 - r4
                t4 = d4 - s4
                t5 = d5 - c5
                return (a[0] + t1 * t1, a[1] + c1 * c1, a[2] + d1 * d1,
                        a[3] + u * u, a[4] + v * v, a[5] + w2 * w2,
                        a[6] + x1 * x1, a[7] + x2 * x2, a[8] + x3 * x3,
                        a[9] + t3 * t3, a[10] + s3 * s3, a[11] + d3 * d3,
                        a[12] + t4 * t4, a[13] + s4 * s4, a[14] + d4 * d4,
                        a[15] + t5 * t5, a[16] + c5 * c5, a[17] + d5 * d5)

            zero = jnp.zeros((16,), jnp.float32)
            a = lax.fori_loop(0, EMB, d_body, (zero,) * 18)

            col64 = jnp.full((16,), EMB, dtype=jnp.int32)
            rad = lambda j: jnp.abs(plsc.load_gather(bufs.at[j], [rows, col64]))
            rc1, rd1 = rad(0), rad(1)
            rc2, rd2 = rad(2), rad(3)
            rc3, rd3 = rad(5), rad(6)
            rc4, rd4 = rad(7), rad(8)
            rc5, rd5 = rad(9), rad(10)

            one = jnp.float32(1.0)
            loss1 = (_relu(_sqrt(a[0]) + rc1 - rd1)
                     + jnp.abs(_sqrt(a[1]) - one) + jnp.abs(_sqrt(a[2]) - one))
            sr2 = rc2 + rd2
            loss2 = (_relu(_sqrt(a[3]) - sr2) + _relu(_sqrt(a[4]) - rc2)
                     + _relu(_sqrt(a[5]) - rd2)
                     + _relu(jnp.maximum(rc2, rd2) - rd2)
                     + jnp.abs(_sqrt(a[6]) - one) + jnp.abs(_sqrt(a[7]) - one)
                     + jnp.abs(_sqrt(a[8]) - one))
            loss3 = (_relu(_sqrt(a[9]) + rc3 - rd3)
                     + jnp.abs(_sqrt(a[10]) - one) + jnp.abs(_sqrt(a[11]) - one))
            loss4 = (_relu(_sqrt(a[12]) - (rc4 + rd4))
                     + jnp.abs(_sqrt(a[13]) - one) + jnp.abs(_sqrt(a[14]) - one))
            loss5 = (_relu(rc5 + rd5 - _sqrt(a[15]) - jnp.float32(0.1))
                     + jnp.abs(_sqrt(a[16]) - one) + jnp.abs(_sqrt(a[17]) - one))
            total = loss1 + loss2 + loss3 + loss4 + loss5
            plsc.store_scatter(out_v, [rows], total)
            return 0

        lax.fori_loop(0, NGRP, group_body, 0)
        pltpu.sync_copy(out_v, out_hbm.at[pl.ds(base, CHUNK)])
        return 0

    lax.fori_loop(0, NCHUNK, chunk_body, 0)


@jax.jit
def _run(cls_pad, rel_view, idx_all):
    mesh = plsc.VectorSubcoreMesh(core_axis_name="c", subcore_axis_name="s")
    scratch = [
        pltpu.VMEM((13, CHUNK), jnp.int32),
        pltpu.VMEM((13, CHUNK, PITCH), jnp.float32),
        pltpu.VMEM((CHUNK,), jnp.float32),
        pltpu.SemaphoreType.DMA,
    ]
    return pl.kernel(
        _body,
        out_type=jax.ShapeDtypeStruct((B,), jnp.float32),
        mesh=mesh,
        scratch_types=scratch,
        compiler_params=pltpu.CompilerParams(
            needs_layout_passes=False, use_tc_tiling_on_sc=True),
    )(cls_pad, rel_view, idx_all)


def kernel(nf1, nf2, nf3, nf4, dis, cls_emb, rel_emb):
    cls_pad = _pad_tbl(cls_emb, EMB + 1)
    rel_view = _pad_tbl(rel_emb, EMB)
    idx_all = jnp.stack([
        nf1[:, 0], nf1[:, 1],
        nf2[:, 0], nf2[:, 1], nf2[:, 2],
        nf3[:, 0], nf3[:, 2],
        nf4[:, 1], nf4[:, 2],
        dis[:, 0], dis[:, 1],
        nf3[:, 1], nf4[:, 0],
    ], axis=0)
    return _run(cls_pad, rel_view, idx_all).reshape(B, 1)


# speedup vs baseline: 1.4414x; 1.4414x over previous
import jax
import jax.numpy as jnp
from jax import lax
from jax.experimental import pallas as pl
from jax.experimental.pallas import tpu as pltpu
from jax.experimental.pallas import tpu_sc as plsc

EMB = 64
B = 16384
PITCH = 128
NB = 1000000

NC = 2
NS = 16
NW = NC * NS
ROWS_PER_W = B // NW
CHUNK = 64
NCHUNK = ROWS_PER_W // CHUNK
NGRP = CHUNK // 16

PAD_R = 4000


def _pad_body(in_ref, out_ref):
    x = in_ref[...]
    out_ref[...] = jnp.pad(x, ((0, 0), (0, PITCH - (EMB + 1))))


def _pad_cls(cls_emb):
    return pl.pallas_call(
        _pad_body,
        grid=(NB // PAD_R,),
        in_specs=[pl.BlockSpec((PAD_R, EMB + 1), lambda i: (i, 0))],
        out_specs=pl.BlockSpec((PAD_R, PITCH), lambda i: (i, 0)),
        out_shape=jax.ShapeDtypeStruct((NB, PITCH), jnp.float32),
    )(cls_emb)


def _sqrt(x):
    xs = jnp.maximum(x, jnp.float32(1e-30))
    i = lax.bitcast_convert_type(xs, jnp.int32)
    i = jnp.int32(0x5F3759DF) - (i >> 1)
    y = lax.bitcast_convert_type(i, jnp.float32)
    for _ in range(3):
        y = y * (jnp.float32(1.5) - jnp.float32(0.5) * xs * y * y)
    return xs * y


def _relu(x):
    return jnp.maximum(x, jnp.float32(0.0))


def _body(cls_hbm, rel_hbm, idx_hbm, out_hbm, idx_v, ridx, bufs, out_v, sem):
    wid = lax.axis_index("s") * NC + lax.axis_index("c")
    iota = lax.iota(jnp.int32, 16)

    def chunk_body(k, _):
        base = wid * ROWS_PER_W + k * CHUNK
        ih = [pltpu.async_copy(idx_hbm.at[j, pl.ds(base, CHUNK)],
                               idx_v.at[j], sem) for j in range(13)]
        for h in ih:
            h.wait()

        def ridx_grp(g, _):
            rows = g * 16 + iota
            for j in range(2):
                idxg = idx_v[11 + j, pl.ds(g * 16, 16)]
                plsc.store_scatter(ridx.at[j], [rows], idxg >> 1)
            return 0

        lax.fori_loop(0, NGRP, ridx_grp, 0)

        handles = []
        for j in range(11):
            handles.append(pltpu.async_copy(
                cls_hbm.at[idx_v.at[j]], bufs.at[j], sem))
        handles.append(pltpu.async_copy(rel_hbm.at[ridx.at[0]], bufs.at[11], sem))
        handles.append(pltpu.async_copy(rel_hbm.at[ridx.at[1]], bufs.at[12], sem))
        for h in handles:
            h.wait()

        def group_body(g, _):
            rows = g * 16 + iota
            ro = [(idx_v[11 + j, pl.ds(g * 16, 16)] & 1) << 6 for j in range(2)]

            def d_body(dd, a):
                col = jnp.full((16,), dd, dtype=jnp.int32)
                ld = lambda j: plsc.load_gather(bufs.at[j], [rows, col])
                c1, d1 = ld(0), ld(1)
                x1, x2, x3 = ld(2), ld(3), ld(4)
                c3, d3 = ld(5), ld(6)
                c4, d4 = ld(7), ld(8)
                c5, d5 = ld(9), ld(10)
                r3 = plsc.load_gather(bufs.at[11], [rows, ro[0] + dd])
                r4 = plsc.load_gather(bufs.at[12], [rows, ro[1] + dd])
                t1 = c1 - d1
                u, v, w2 = x2 - x1, x3 - x1, x3 - x2
                s3 = c3 + r3
                t3 = s3 - d3
                s4 = c4 - r4
                t4 = d4 - s4
                t5 = d5 - c5
                return (a[0] + t1 * t1, a[1] + c1 * c1, a[2] + d1 * d1,
                        a[3] + u * u, a[4] + v * v, a[5] + w2 * w2,
                        a[6] + x1 * x1, a[7] + x2 * x2, a[8] + x3 * x3,
                        a[9] + t3 * t3, a[10] + s3 * s3, a[11] + d3 * d3,
                        a[12] + t4 * t4, a[13] + s4 * s4, a[14] + d4 * d4,
                        a[15] + t5 * t5, a[16] + c5 * c5, a[17] + d5 * d5)

            zero = jnp.zeros((16,), jnp.float32)
            a = lax.fori_loop(0, EMB, d_body, (zero,) * 18)

            col64 = jnp.full((16,), EMB, dtype=jnp.int32)
            rad = lambda j: jnp.abs(plsc.load_gather(bufs.at[j], [rows, col64]))
            rc1, rd1 = rad(0), rad(1)
            rc2, rd2 = rad(2), rad(3)
            rc3, rd3 = rad(5), rad(6)
            rc4, rd4 = rad(7), rad(8)
            rc5, rd5 = rad(9), rad(10)

            one = jnp.float32(1.0)
            loss1 = (_relu(_sqrt(a[0]) + rc1 - rd1)
                     + jnp.abs(_sqrt(a[1]) - one) + jnp.abs(_sqrt(a[2]) - one))
            sr2 = rc2 + rd2
            loss2 = (_relu(_sqrt(a[3]) - sr2) + _relu(_sqrt(a[4]) - rc2)
                     + _relu(_sqrt(a[5]) - rd2)
                     + _relu(jnp.maximum(rc2, rd2) - rd2)
                     + jnp.abs(_sqrt(a[6]) - one) + jnp.abs(_sqrt(a[7]) - one)
                     + jnp.abs(_sqrt(a[8]) - one))
            loss3 = (_relu(_sqrt(a[9]) + rc3 - rd3)
                     + jnp.abs(_sqrt(a[10]) - one) + jnp.abs(_sqrt(a[11]) - one))
            loss4 = (_relu(_sqrt(a[12]) - (rc4 + rd4))
                     + jnp.abs(_sqrt(a[13]) - one) + jnp.abs(_sqrt(a[14]) - one))
            loss5 = (_relu(rc5 + rd5 - _sqrt(a[15]) - jnp.float32(0.1))
                     + jnp.abs(_sqrt(a[16]) - one) + jnp.abs(_sqrt(a[17]) - one))
            total = loss1 + loss2 + loss3 + loss4 + loss5
            plsc.store_scatter(out_v, [rows], total)
            return 0

        lax.fori_loop(0, NGRP, group_body, 0)
        pltpu.sync_copy(out_v, out_hbm.at[pl.ds(base, CHUNK)])
        return 0

    lax.fori_loop(0, NCHUNK, chunk_body, 0)


@jax.jit
def _run(cls_pad, rel_view, idx_all):
    mesh = plsc.VectorSubcoreMesh(core_axis_name="c", subcore_axis_name="s")
    scratch = [
        pltpu.VMEM((13, CHUNK), jnp.int32),
        pltpu.VMEM((2, CHUNK), jnp.int32),
        pltpu.VMEM((13, CHUNK, PITCH), jnp.float32),
        pltpu.VMEM((CHUNK,), jnp.float32),
        pltpu.SemaphoreType.DMA,
    ]
    return pl.kernel(
        _body,
        out_type=jax.ShapeDtypeStruct((B,), jnp.float32),
        mesh=mesh,
        scratch_types=scratch,
        compiler_params=pltpu.CompilerParams(
            needs_layout_passes=False, use_tc_tiling_on_sc=True),
    )(cls_pad, rel_view, idx_all)


def kernel(nf1, nf2, nf3, nf4, dis, cls_emb, rel_emb):
    cls_pad = _pad_cls(cls_emb)
    rel_view = rel_emb.reshape(NB * EMB // PITCH, PITCH)
    idx_all = jnp.stack([
        nf1[:, 0], nf1[:, 1],
        nf2[:, 0], nf2[:, 1], nf2[:, 2],
        nf3[:, 0], nf3[:, 2],
        nf4[:, 1], nf4[:, 2],
        dis[:, 0], dis[:, 1],
        nf3[:, 1], nf4[:, 0],
    ], axis=0)
    return _run(cls_pad, rel_view, idx_all).reshape(B, 1)


# speedup vs baseline: 1.4728x; 1.0218x over previous
import jax
import jax.numpy as jnp
from jax import lax
from jax.experimental import pallas as pl
from jax.experimental.pallas import tpu as pltpu
from jax.experimental.pallas import tpu_sc as plsc

EMB = 64
B = 16384
PITCH = 128
NB = 1000000

NC = 2
NS = 16
NW = NC * NS
ROWS_PER_W = B // NW
CHUNK = 64
NCHUNK = ROWS_PER_W // CHUNK
NGRP = CHUNK // 16

PAD_R = 10000


def _pad_body(in_ref, out_ref):
    out_ref[:, : EMB + 1] = in_ref[...]


def _pad_cls(cls_emb):
    return pl.pallas_call(
        _pad_body,
        grid=(NB // PAD_R,),
        in_specs=[pl.BlockSpec((PAD_R, EMB + 1), lambda i: (i, 0))],
        out_specs=pl.BlockSpec((PAD_R, PITCH), lambda i: (i, 0)),
        out_shape=jax.ShapeDtypeStruct((NB, PITCH), jnp.float32),
    )(cls_emb)


def _sqrt(x):
    xs = jnp.maximum(x, jnp.float32(1e-30))
    i = lax.bitcast_convert_type(xs, jnp.int32)
    i = jnp.int32(0x5F3759DF) - (i >> 1)
    y = lax.bitcast_convert_type(i, jnp.float32)
    for _ in range(3):
        y = y * (jnp.float32(1.5) - jnp.float32(0.5) * xs * y * y)
    return xs * y


def _relu(x):
    return jnp.maximum(x, jnp.float32(0.0))


def _body(cls_hbm, rel_hbm, idx_hbm, out_hbm, idx_v, ridx, bufs, out_v, sem):
    wid = lax.axis_index("s") * NC + lax.axis_index("c")
    iota = lax.iota(jnp.int32, 16)

    def chunk_body(k, _):
        base = wid * ROWS_PER_W + k * CHUNK
        ih = [pltpu.async_copy(idx_hbm.at[j, pl.ds(base, CHUNK)],
                               idx_v.at[j], sem) for j in range(13)]
        for h in ih:
            h.wait()

        def ridx_grp(g, _):
            rows = g * 16 + iota
            for j in range(2):
                idxg = idx_v[11 + j, pl.ds(g * 16, 16)]
                plsc.store_scatter(ridx.at[j], [rows], idxg >> 1)
            return 0

        lax.fori_loop(0, NGRP, ridx_grp, 0)

        handles = []
        for j in range(11):
            handles.append(pltpu.async_copy(
                cls_hbm.at[idx_v.at[j]], bufs.at[j], sem))
        handles.append(pltpu.async_copy(rel_hbm.at[ridx.at[0]], bufs.at[11], sem))
        handles.append(pltpu.async_copy(rel_hbm.at[ridx.at[1]], bufs.at[12], sem))
        for h in handles:
            h.wait()

        def group_body(g, _):
            rows = g * 16 + iota
            ro = [(idx_v[11 + j, pl.ds(g * 16, 16)] & 1) << 6 for j in range(2)]

            def d_body(dd, a):
                col = jnp.full((16,), dd, dtype=jnp.int32)
                ld = lambda j: plsc.load_gather(bufs.at[j], [rows, col])
                c1, d1 = ld(0), ld(1)
                x1, x2, x3 = ld(2), ld(3), ld(4)
                c3, d3 = ld(5), ld(6)
                c4, d4 = ld(7), ld(8)
                c5, d5 = ld(9), ld(10)
                r3 = plsc.load_gather(bufs.at[11], [rows, ro[0] + dd])
                r4 = plsc.load_gather(bufs.at[12], [rows, ro[1] + dd])
                t1 = c1 - d1
                u, v, w2 = x2 - x1, x3 - x1, x3 - x2
                s3 = c3 + r3
                t3 = s3 - d3
                s4 = c4 - r4
                t4 = d4 - s4
                t5 = d5 - c5
                return (a[0] + t1 * t1, a[1] + c1 * c1, a[2] + d1 * d1,
                        a[3] + u * u, a[4] + v * v, a[5] + w2 * w2,
                        a[6] + x1 * x1, a[7] + x2 * x2, a[8] + x3 * x3,
                        a[9] + t3 * t3, a[10] + s3 * s3, a[11] + d3 * d3,
                        a[12] + t4 * t4, a[13] + s4 * s4, a[14] + d4 * d4,
                        a[15] + t5 * t5, a[16] + c5 * c5, a[17] + d5 * d5)

            zero = jnp.zeros((16,), jnp.float32)
            a = lax.fori_loop(0, EMB, d_body, (zero,) * 18)

            col64 = jnp.full((16,), EMB, dtype=jnp.int32)
            rad = lambda j: jnp.abs(plsc.load_gather(bufs.at[j], [rows, col64]))
            rc1, rd1 = rad(0), rad(1)
            rc2, rd2 = rad(2), rad(3)
            rc3, rd3 = rad(5), rad(6)
            rc4, rd4 = rad(7), rad(8)
            rc5, rd5 = rad(9), rad(10)

            one = jnp.float32(1.0)
            loss1 = (_relu(_sqrt(a[0]) + rc1 - rd1)
                     + jnp.abs(_sqrt(a[1]) - one) + jnp.abs(_sqrt(a[2]) - one))
            sr2 = rc2 + rd2
            loss2 = (_relu(_sqrt(a[3]) - sr2) + _relu(_sqrt(a[4]) - rc2)
                     + _relu(_sqrt(a[5]) - rd2)
                     + _relu(jnp.maximum(rc2, rd2) - rd2)
                     + jnp.abs(_sqrt(a[6]) - one) + jnp.abs(_sqrt(a[7]) - one)
                     + jnp.abs(_sqrt(a[8]) - one))
            loss3 = (_relu(_sqrt(a[9]) + rc3 - rd3)
                     + jnp.abs(_sqrt(a[10]) - one) + jnp.abs(_sqrt(a[11]) - one))
            loss4 = (_relu(_sqrt(a[12]) - (rc4 + rd4))
                     + jnp.abs(_sqrt(a[13]) - one) + jnp.abs(_sqrt(a[14]) - one))
            loss5 = (_relu(rc5 + rd5 - _sqrt(a[15]) - jnp.float32(0.1))
                     + jnp.abs(_sqrt(a[16]) - one) + jnp.abs(_sqrt(a[17]) - one))
            total = loss1 + loss2 + loss3 + loss4 + loss5
            plsc.store_scatter(out_v, [rows], total)
            return 0

        lax.fori_loop(0, NGRP, group_body, 0)
        pltpu.sync_copy(out_v, out_hbm.at[pl.ds(base, CHUNK)])
        return 0

    lax.fori_loop(0, NCHUNK, chunk_body, 0)


@jax.jit
def _run(cls_pad, rel_view, idx_all):
    mesh = plsc.VectorSubcoreMesh(core_axis_name="c", subcore_axis_name="s")
    scratch = [
        pltpu.VMEM((13, CHUNK), jnp.int32),
        pltpu.VMEM((2, CHUNK), jnp.int32),
        pltpu.VMEM((13, CHUNK, PITCH), jnp.float32),
        pltpu.VMEM((CHUNK,), jnp.float32),
        pltpu.SemaphoreType.DMA,
    ]
    return pl.kernel(
        _body,
        out_type=jax.ShapeDtypeStruct((B,), jnp.float32),
        mesh=mesh,
        scratch_types=scratch,
        compiler_params=pltpu.CompilerParams(
            needs_layout_passes=False, use_tc_tiling_on_sc=True),
    )(cls_pad, rel_view, idx_all)


def kernel(nf1, nf2, nf3, nf4, dis, cls_emb, rel_emb):
    cls_pad = _pad_cls(cls_emb)
    rel_view = rel_emb.reshape(NB * EMB // PITCH, PITCH)
    idx_all = jnp.stack([
        nf1[:, 0], nf1[:, 1],
        nf2[:, 0], nf2[:, 1], nf2[:, 2],
        nf3[:, 0], nf3[:, 2],
        nf4[:, 1], nf4[:, 2],
        dis[:, 0], dis[:, 1],
        nf3[:, 1], nf4[:, 0],
    ], axis=0)
    return _run(cls_pad, rel_view, idx_all).reshape(B, 1)
